# Initial kernel scaffold; baseline (speedup 1.0000x reference)
import jax
import jax.numpy as jnp
from jax.experimental import pallas as pl


def kernel(x, edge_index, batch, Wl1, bl1, Wr1, wp1, Wl2, bl2, Wr2, wp2, W1, b1, W2, b2):
    raise NotImplementedError("write your pallas kernel here")


# speedup vs baseline: 23.1941x; 23.1941x over previous
import jax
import jax.numpy as jnp
import numpy as np
from jax import lax
from jax.experimental import pallas as pl
from jax.experimental.pallas import tpu as pltpu
from jax.experimental.pallas import tpu_sc as plsc

N = 50000
E = 800000
R = 50176
CH = 128
NCH = 6400
EPAD = NCH * CH
NS = 16
TROWS = R // NS
ZR = 112
IB = 8
NBUF = 2
K1 = 40000
K2 = 32000


def _sortable_const(v):
    b = np.float32(v).view(np.int32)
    return int(b ^ ((b >> 31) & np.int32(0x7FFFFFFF)))


_KLO = _sortable_const(-2.0)
_KHI = _sortable_const(1.0) + 1


def _make_sc_agg(ncols, split_edges, gl, ibg):
    nblk = (EPAD // 32 if split_edges else EPAD // NS) // (gl * ibg)
    mesh = plsc.VectorSubcoreMesh(core_axis_name="c", subcore_axis_name="s")

    def body(tab, src3, dst3, out, src_v, dst_v, rows_v, zbuf, acc,
             gsem, ssem):
        c = lax.axis_index("c")
        s = lax.axis_index("s")

        z16 = jnp.zeros((16,), jnp.float32)

        def zrow(i, _):
            for c0 in range(0, ncols, 16):
                zbuf[i, pl.ds(c0, 16)] = z16
            return 0

        lax.fori_loop(0, ZR, zrow, 0)

        def zacc(t, _):
            pltpu.sync_copy(zbuf, acc.at[pl.ds(s * TROWS + t * ZR, ZR)])
            return 0

        lax.fori_loop(0, TROWS // ZR, zacc, 0)
        plsc.subcore_barrier()

        if split_edges:
            base_blk = (c * NS + s) * nblk
        else:
            base_blk = s * nblk

        def block(b, _):
            blk = base_blk + b
            pltpu.sync_copy(src3.at[blk], src_v)
            pltpu.sync_copy(dst3.at[blk], dst_v)

            def start_gather(j, q):
                idx = src_v.at[j]
                if split_edges:
                    return pltpu.async_copy(
                        tab.at[0].at[idx], rows_v.at[q], gsem.at[q])

                @pl.when(c == 0)
                def _():
                    pltpu.async_copy(
                        tab.at[0].at[idx], rows_v.at[q], gsem.at[q])

                @pl.when(c == 1)
                def _():
                    pltpu.async_copy(
                        tab.at[1].at[idx], rows_v.at[q], gsem.at[q])
                return pltpu.make_async_copy(
                    tab.at[0].at[idx], rows_v.at[q], gsem.at[q])

            def start_scatter(j, q):
                return pltpu.async_copy(
                    rows_v.at[q], acc.at[dst_v.at[j]], ssem.at[q], add=True)

            gds = {}
            sds = {}
            for j in range(ibg):
                q = j % NBUF
                if j >= NBUF:
                    sds[j - NBUF].wait()
                gds[j] = start_gather(j, q)
                if j >= 1:
                    gds[j - 1].wait()
                    sds[j - 1] = start_scatter(j - 1, (j - 1) % NBUF)
            gds[ibg - 1].wait()
            sds[ibg - 1] = start_scatter(ibg - 1, (ibg - 1) % NBUF)
            for j in range(max(0, ibg - NBUF), ibg):
                sds[j].wait()
            return 0

        lax.fori_loop(0, nblk, block, 0)
        plsc.subcore_barrier()

        def wout(t, _):
            sl = pl.ds(s * TROWS + t * ZR, ZR)
            pltpu.sync_copy(acc.at[sl], out.at[c].at[sl])
            return 0

        lax.fori_loop(0, TROWS // ZR, wout, 0)

    return pl.kernel(
        body,
        out_type=jax.ShapeDtypeStruct((2, R, ncols), jnp.float32),
        mesh=mesh,
        compiler_params=pltpu.CompilerParams(use_tc_tiling_on_sc=False),
        scratch_types=[
            pltpu.VMEM((ibg, gl), jnp.int32),
            pltpu.VMEM((ibg, gl), jnp.int32),
            pltpu.VMEM((NBUF, gl, ncols), jnp.float32),
            pltpu.VMEM((ZR, ncols), jnp.float32),
            pltpu.VMEM_SHARED((R, ncols), jnp.float32),
            pltpu.SemaphoreType.DMA((NBUF,)),
            pltpu.SemaphoreType.DMA((NBUF,)),
        ],
    )


_sc_l1 = _make_sc_agg(32, True, 256, 4)
_sc_l2f = _make_sc_agg(32, False, 256, 8)
_sc_l2c = _make_sc_agg(16, True, 512, 2)


BR = 1024
NB = R // BR
SR = BR // 128


def _keys_of(score):
    b = lax.bitcast_convert_type(score, jnp.int32)
    return b ^ ((b >> 31) & jnp.int32(0x7FFFFFFF))


def _score_block(h, wp, i):
    inv = lax.rsqrt(jnp.sum(wp * wp))
    s = jnp.tanh(jnp.dot(h, wp, preferred_element_type=jnp.float32) * inv)
    rows = lax.broadcasted_iota(jnp.int32, (BR, 1), 0) + i * BR
    return jnp.where(rows < N, s, -2.0), rows


def _kept_block(s, rows, thr, jstar):
    keys = _keys_of(s)
    return (keys > thr) | ((keys == thr) & (rows < jstar))


def _bisect(scores, k):
    keys = _keys_of(scores)
    flat = (lax.broadcasted_iota(jnp.int32, (R // 128, 128), 0) * 128
            + lax.broadcasted_iota(jnp.int32, (R // 128, 128), 1))

    def cge(t):
        return jnp.sum((keys >= t).astype(jnp.int32))

    def bis(_, lh):
        lo, hi = lh
        mid = lo + (hi - lo) // 2
        big = cge(mid) >= k
        return jnp.where(big, mid, lo), jnp.where(big, hi, mid)

    lo, _ = lax.fori_loop(
        0, 32, bis, (jnp.int32(_KLO), jnp.int32(_KHI)))
    need = k - cge(lo + 1)

    def ctie(j):
        return jnp.sum(((keys == lo) & (flat < j)).astype(jnp.int32))

    def bis2(_, lh):
        lo2, hi2 = lh
        mid = (lo2 + hi2) // 2
        big = ctie(mid) >= need
        return jnp.where(big, lo2, mid + 1), jnp.where(big, mid, hi2)

    _, jstar = lax.fori_loop(0, 17, bis2, (jnp.int32(0), jnp.int32(R)))
    return lo, jstar


def _tc1a(xp_ref, parts_ref, wl1t_ref, wr1t_ref, bl1_ref, wp1_ref,
          h_ref, sc_ref):
    i = pl.program_id(0)
    agg = parts_ref[0] + parts_ref[1]
    mean1 = agg[:, :24] / jnp.maximum(agg[:, 24:25], 1.0)
    h = jnp.dot(mean1, wl1t_ref[...], preferred_element_type=jnp.float32)
    h = h + jnp.dot(xp_ref[...], wr1t_ref[...],
                    preferred_element_type=jnp.float32)
    h = jnp.maximum(h + bl1_ref[...], 0.0)
    h_ref[...] = h
    s, _ = _score_block(h, wp1_ref[...], i)
    sc_ref[...] = s.reshape(1, SR, 128)


def _tc1c(h_ref, wp1_ref, sc_ref, gcat_ref, tc_ref, x1_ref, tk_smem):
    i = pl.program_id(0)

    @pl.when(i == 0)
    def _():
        thr0, jstar0 = _bisect(sc_ref[...], K1)
        tk_smem[0] = thr0
        tk_smem[1] = jstar0

    h = h_ref[...]
    s, rows = _score_block(h, wp1_ref[...], i)
    kept = _kept_block(s, rows, tk_smem[0], tk_smem[1])
    g = h * jnp.where(kept, s, 0.0)
    gcat_ref[0] = g[:, :32]
    gcat_ref[1] = g[:, 32:]
    tc_ref[...] = jnp.concatenate(
        [kept.astype(jnp.float32), jnp.zeros((BR, 15), jnp.float32)], axis=1)
    bmax = jnp.max(jnp.where(kept, g, -jnp.inf), axis=0, keepdims=True)
    bsum = jnp.sum(g, axis=0, keepdims=True) * (1.0 / K1)

    @pl.when(i == 0)
    def _():
        x1_ref[0:1, :] = bmax
        x1_ref[1:2, :] = bsum

    @pl.when(i > 0)
    def _():
        x1_ref[0:1, :] = jnp.maximum(x1_ref[0:1, :], bmax)
        x1_ref[1:2, :] = x1_ref[1:2, :] + bsum


def _tc2a(a_ref, cp_ref, g_ref, tc_ref, wl2t_ref, wr2t_ref, bl2_ref,
          wp2_ref, h2_ref, sc2_ref):
    i = pl.program_id(0)
    agg2 = jnp.concatenate([a_ref[0], a_ref[1]], axis=1)
    cnt2 = cp_ref[0][:, 0:1] + cp_ref[1][:, 0:1]
    g = jnp.concatenate([g_ref[0], g_ref[1]], axis=1)
    mean2 = agg2 / jnp.maximum(cnt2, 1.0)
    h2 = jnp.dot(mean2, wl2t_ref[...], preferred_element_type=jnp.float32)
    h2 = h2 + jnp.dot(g, wr2t_ref[...], preferred_element_type=jnp.float32)
    h2 = jnp.maximum(h2 + bl2_ref[...], 0.0)
    h2_ref[...] = h2
    s2, _ = _score_block(h2, wp2_ref[...], i)
    kf = tc_ref[:, 0:1]
    s2 = jnp.where(kf > 0.0, s2, -2.0)
    sc2_ref[...] = s2.reshape(1, SR, 128)


def _tc2c(h2_ref, tc_ref, wp2_ref, sc_ref, x1_ref,
          w1t_ref, b1_ref, w2t_ref, b2_ref, out_ref, x2_acc, tk_smem):
    i = pl.program_id(0)

    @pl.when(i == 0)
    def _():
        thr0, jstar0 = _bisect(sc_ref[...], K2)
        tk_smem[0] = thr0
        tk_smem[1] = jstar0

    h2 = h2_ref[...]
    s2, rows = _score_block(h2, wp2_ref[...], i)
    kf = tc_ref[:, 0:1]
    s2 = jnp.where(kf > 0.0, s2, -2.0)
    kept2 = _kept_block(s2, rows, tk_smem[0], tk_smem[1])
    g2 = h2 * jnp.where(kept2, s2, 0.0)
    bmax = jnp.max(jnp.where(kept2, g2, -jnp.inf), axis=0, keepdims=True)
    bsum = jnp.sum(g2, axis=0, keepdims=True) * (1.0 / K2)

    @pl.when(i == 0)
    def _():
        x2_acc[0:1, :] = bmax
        x2_acc[1:2, :] = bsum

    @pl.when(i > 0)
    def _():
        x2_acc[0:1, :] = jnp.maximum(x2_acc[0:1, :], bmax)
        x2_acc[1:2, :] = x2_acc[1:2, :] + bsum

    @pl.when(i == NB - 1)
    def _():
        x1v = x1_ref[...]
        x2v = x2_acc[...]
        z = jnp.concatenate(
            [x1v[0:1, :] + x2v[0:1, :], x1v[1:2, :] + x2v[1:2, :]], axis=1)
        z = jnp.maximum(
            jnp.dot(z, w1t_ref[...], preferred_element_type=jnp.float32)
            + b1_ref[...], 0.0)
        out_ref[...] = (
            jnp.dot(z, w2t_ref[...], preferred_element_type=jnp.float32)
            + b2_ref[...])


def _full(shape):
    return pl.BlockSpec(shape, lambda i: (0,) * len(shape))


_tc1a_call = pl.pallas_call(
    _tc1a,
    grid=(NB,),
    in_specs=[
        pl.BlockSpec((BR, 24), lambda i: (i, 0)),
        pl.BlockSpec((2, BR, 32), lambda i: (0, i, 0)),
        _full((24, 64)), _full((24, 64)), _full((1, 64)), _full((64, 1)),
    ],
    out_specs=[
        pl.BlockSpec((BR, 64), lambda i: (i, 0)),
        pl.BlockSpec((1, SR, 128), lambda i: (i, 0, 0)),
    ],
    out_shape=[
        jax.ShapeDtypeStruct((R, 64), jnp.float32),
        jax.ShapeDtypeStruct((NB, SR, 128), jnp.float32),
    ],
)


_tc1c_call = pl.pallas_call(
    _tc1c,
    grid=(NB,),
    in_specs=[
        pl.BlockSpec((BR, 64), lambda i: (i, 0)),
        _full((64, 1)), _full((R // 128, 128)),
    ],
    scratch_shapes=[pltpu.SMEM((2,), jnp.int32)],
    out_specs=[
        pl.BlockSpec((2, BR, 32), lambda i: (0, i, 0)),
        pl.BlockSpec((BR, 16), lambda i: (i, 0)),
        _full((2, 64)),
    ],
    out_shape=[
        jax.ShapeDtypeStruct((2, R, 32), jnp.float32),
        jax.ShapeDtypeStruct((R, 16), jnp.float32),
        jax.ShapeDtypeStruct((2, 64), jnp.float32),
    ],
)

_tc2a_call = pl.pallas_call(
    _tc2a,
    grid=(NB,),
    in_specs=[
        pl.BlockSpec((2, BR, 32), lambda i: (0, i, 0)),
        pl.BlockSpec((2, BR, 16), lambda i: (0, i, 0)),
        pl.BlockSpec((2, BR, 32), lambda i: (0, i, 0)),
        pl.BlockSpec((BR, 16), lambda i: (i, 0)),
        _full((64, 64)), _full((64, 64)), _full((1, 64)), _full((64, 1)),
    ],
    out_specs=[
        pl.BlockSpec((BR, 64), lambda i: (i, 0)),
        pl.BlockSpec((1, SR, 128), lambda i: (i, 0, 0)),
    ],
    out_shape=[
        jax.ShapeDtypeStruct((R, 64), jnp.float32),
        jax.ShapeDtypeStruct((NB, SR, 128), jnp.float32),
    ],
)

_tc2c_call = pl.pallas_call(
    _tc2c,
    grid=(NB,),
    in_specs=[
        pl.BlockSpec((BR, 64), lambda i: (i, 0)),
        pl.BlockSpec((BR, 16), lambda i: (i, 0)),
        _full((64, 1)), _full((R // 128, 128)), _full((2, 64)),
        _full((128, 64)), _full((1, 64)), _full((64, 12)), _full((1, 12)),
    ],
    out_specs=[_full((1, 12))],
    out_shape=[jax.ShapeDtypeStruct((1, 12), jnp.float32)],
    scratch_shapes=[pltpu.VMEM((2, 64), jnp.float32),
                    pltpu.SMEM((2,), jnp.int32)],
)


@jax.jit
def kernel(x, edge_index, batch, Wl1, bl1, Wr1, wp1,
           Wl2, bl2, Wr2, wp2, W1, b1, W2, b2):
    del batch
    f32 = jnp.float32
    pad = jnp.full((EPAD - E,), N, jnp.int32)
    srcf = jnp.concatenate([edge_index[0], pad])
    dstf = jnp.concatenate([edge_index[1], pad])

    x32 = jnp.zeros((R, 32), f32).at[:N, :24].set(x).at[:N, 24].set(1.0)
    xp = jnp.zeros((R, 24), f32).at[:N].set(x)

    parts1 = _sc_l1(x32[None], srcf.reshape(800, 4, 256),
                    dstf.reshape(800, 4, 256))

    h, sc1 = _tc1a_call(xp, parts1, Wl1.T, Wr1.T,
                        bl1.reshape(1, 64), wp1.reshape(64, 1))
    gcat, tc_tab, x1 = _tc1c_call(
        h, wp1.reshape(64, 1), sc1.reshape(R // 128, 128))

    agg2 = _sc_l2f(gcat, srcf.reshape(400, 8, 256),
                   dstf.reshape(400, 8, 256))
    cparts = _sc_l2c(tc_tab[None], srcf.reshape(800, 2, 512),
                     dstf.reshape(800, 2, 512))

    h2, sc2 = _tc2a_call(agg2, cparts, gcat, tc_tab, Wl2.T, Wr2.T,
                         bl2.reshape(1, 64), wp2.reshape(64, 1))
    (out,) = _tc2c_call(h2, tc_tab, wp2.reshape(64, 1),
                        sc2.reshape(R // 128, 128), x1,
                        W1.T, b1.reshape(1, 64), W2.T, b2.reshape(1, 12))
    return out
